# Initial kernel scaffold; baseline (speedup 1.0000x reference)
import functools

import jax
import jax.numpy as jnp
from jax import lax
from jax.experimental import pallas as pl
from jax.experimental.pallas import tpu as pltpu
from jax.experimental.pallas import tpu_sc as plsc

NC = 2
NS = 16
NW = NC * NS
B = 4096
D = 128
N_OBJ = 64
B_PER_W = B // NW
THRESHOLD = 4.0


def _valuation_body(zs_hbm, ti_hbm, out_hbm, ti_v, ia_v, ib_v, a_v, b_v, o_v,
                    sem_a, sem_b):
    wid = lax.axis_index("s") * NC + lax.axis_index("c")
    base = wid * B_PER_W

    pltpu.sync_copy(ti_hbm, ti_v)
    lane0 = jnp.zeros((16,), jnp.int32)
    i_vec = plsc.load_gather(ti_v, [lane0])
    j_vec = plsc.load_gather(ti_v, [lane0 + 1])

    iota = lax.iota(jnp.int32, (16,))
    for c in range(B_PER_W // 16):
        r = (base + c * 16 + iota) * N_OBJ
        ia_v[pl.ds(c * 16, 16)] = r + i_vec
        ib_v[pl.ds(c * 16, 16)] = r + j_vec

    cp_a = pltpu.async_copy(zs_hbm.at[ia_v], a_v, sem_a)
    cp_b = pltpu.async_copy(zs_hbm.at[ib_v], b_v, sem_b)
    cp_a.wait()
    cp_b.wait()

    for g in range(B_PER_W // 16):
        rv = g * 16 + iota

        def col_step(k, carry):
            acc, kv = carry
            av = plsc.load_gather(a_v, [rv, kv])
            bv = plsc.load_gather(b_v, [rv, kv])
            d = av - bv
            return acc + d * d, kv + 1

        acc, _ = lax.fori_loop(
            0, D, col_step,
            (jnp.zeros((16,), jnp.float32), jnp.zeros((16,), jnp.int32)),
            unroll=16)

        x = acc + 1e-12
        bits = plsc.bitcast(x, jnp.int32)
        y = plsc.bitcast(jnp.int32(0x5F3759DF) - (bits >> 1), jnp.float32)
        for _ in range(3):
            y = y * (1.5 - 0.5 * x * y * y)
        dist = x * y
        prob = 0.01 + 0.98 / (1.0 + jnp.exp(dist - THRESHOLD))
        o_v[pl.ds(g * 16, 16)] = prob

    pltpu.sync_copy(o_v, out_hbm.at[pl.ds(base, B_PER_W)])


_valuation_sc = functools.partial(
    pl.kernel,
    out_type=jax.ShapeDtypeStruct((B,), jnp.float32),
    mesh=plsc.VectorSubcoreMesh(
        core_axis_name="c", subcore_axis_name="s",
        num_cores=NC, num_subcores=NS),
    scratch_types=[
        pltpu.VMEM((16,), jnp.int32),
        pltpu.VMEM((B_PER_W,), jnp.int32),
        pltpu.VMEM((B_PER_W,), jnp.int32),
        pltpu.VMEM((B_PER_W, D), jnp.float32),
        pltpu.VMEM((B_PER_W, D), jnp.float32),
        pltpu.VMEM((B_PER_W,), jnp.float32),
        pltpu.SemaphoreType.DMA,
        pltpu.SemaphoreType.DMA,
    ],
)(_valuation_body)


def kernel(zs, term_idx):
    zs2 = zs.reshape(B * N_OBJ, D)
    ti = jnp.zeros((16,), jnp.int32).at[:2].set(term_idx.astype(jnp.int32))
    return _valuation_sc(zs2, ti)


# speedup vs baseline: 2.8145x; 2.8145x over previous
import functools

import jax
import jax.numpy as jnp
from jax import lax
from jax.experimental import pallas as pl
from jax.experimental.pallas import tpu as pltpu
from jax.experimental.pallas import tpu_sc as plsc

NC = 2
NS = 16
NW = NC * NS
B = 4096
D = 128
N_OBJ = 64
B_PER_W = B // NW
THRESHOLD = 4.0


def _valuation_body(zs_hbm, ti_hbm, out_hbm, ti_v, ia_v, ib_v, a_v, b_v, o_v,
                    sem_a, sem_b):
    wid = lax.axis_index("s") * NC + lax.axis_index("c")
    base = wid * B_PER_W

    pltpu.sync_copy(ti_hbm, ti_v)
    i_vec = ti_v[pl.ds(0, 16)]
    j_vec = ti_v[pl.ds(16, 16)]

    iota = lax.iota(jnp.int32, 16)
    for c in range(B_PER_W // 16):
        r = (base + c * 16 + iota) * N_OBJ
        ia_v[pl.ds(c * 16, 16)] = r + i_vec
        ib_v[pl.ds(c * 16, 16)] = r + j_vec

    cp_a = pltpu.async_copy(zs_hbm.at[ia_v], a_v, sem_a)
    cp_b = pltpu.async_copy(zs_hbm.at[ib_v], b_v, sem_b)
    cp_a.wait()
    cp_b.wait()

    for g in range(B_PER_W // 16):
        rv = g * 16 + iota

        def col_step(k, carry):
            acc, kv = carry
            av = plsc.load_gather(a_v, [rv, kv])
            bv = plsc.load_gather(b_v, [rv, kv])
            d = av - bv
            return acc + d * d, kv + 1

        acc, _ = lax.fori_loop(
            0, D, col_step,
            (jnp.zeros((16,), jnp.float32), jnp.zeros((16,), jnp.int32)),
            unroll=16)

        x = acc + 1e-12
        bits = plsc.bitcast(x, jnp.int32)
        y = plsc.bitcast(jnp.int32(0x5F3759DF) - (bits >> 1), jnp.float32)
        for _ in range(3):
            y = y * (1.5 - 0.5 * x * y * y)
        dist = x * y
        prob = 0.01 + 0.98 / (1.0 + jnp.exp(dist - THRESHOLD))
        o_v[pl.ds(g * 16, 16)] = prob

    pltpu.sync_copy(o_v, out_hbm.at[pl.ds(base, B_PER_W)])


_valuation_sc = functools.partial(
    pl.kernel,
    out_type=jax.ShapeDtypeStruct((B,), jnp.float32),
    mesh=plsc.VectorSubcoreMesh(
        core_axis_name="c", subcore_axis_name="s",
        num_cores=NC, num_subcores=NS),
    scratch_types=[
        pltpu.VMEM((128,), jnp.int32),
        pltpu.VMEM((B_PER_W,), jnp.int32),
        pltpu.VMEM((B_PER_W,), jnp.int32),
        pltpu.VMEM((B_PER_W, D), jnp.float32),
        pltpu.VMEM((B_PER_W, D), jnp.float32),
        pltpu.VMEM((B_PER_W,), jnp.float32),
        pltpu.SemaphoreType.DMA,
        pltpu.SemaphoreType.DMA,
    ],
    compiler_params=pltpu.CompilerParams(needs_layout_passes=False),
)(_valuation_body)


def kernel(zs, term_idx):
    zs2 = zs.reshape(B * N_OBJ, D)
    ti32 = term_idx.astype(jnp.int32)
    ti = jnp.concatenate([jnp.full((16,), ti32[0], jnp.int32),
                          jnp.full((16,), ti32[1], jnp.int32),
                          jnp.zeros((96,), jnp.int32)])
    return _valuation_sc(zs2, ti)
